# baseline (speedup 1.0000x reference)
import functools

import jax
import jax.numpy as jnp
from jax import lax
from jax.experimental import pallas as pl
from jax.experimental.pallas import tpu as pltpu
from jax.experimental.pallas import tpu_sc as plsc

NC = 2
NS = 16
NW = NC * NS
CH = 128
D = 128


def _sc_agg_body(nch, agg_rows,
                 h_hbm, src_hbm, dst_hbm, out_hbm,
                 sidx0, sidx1, didx0, didx1, msg0, msg1, agg_sh,
                 gsem0, gsem1, ssem0, ssem1, is0, is1, id0, id1):
    c = lax.axis_index("c")
    s = lax.axis_index("s")
    wid = c * NS + s

    sidx = (sidx0, sidx1)
    didx = (didx0, didx1)
    msg = (msg0, msg1)
    gsem = (gsem0, gsem1)
    ssem = (ssem0, ssem1)
    isem = (is0, is1)
    dsem = (id0, id1)

    zrows = agg_rows // NS

    def _zero_row(i, _):
        for j in range(D // 16):
            msg0[i, pl.ds(j * 16, 16)] = jnp.zeros((16,), jnp.float32)
        return 0

    lax.fori_loop(0, CH, _zero_row, 0)
    for k in range(zrows // CH):
        pltpu.sync_copy(msg0, agg_sh.at[pl.ds(s * zrows + k * CH, CH)])
    rem = zrows % CH
    if rem:
        pltpu.sync_copy(msg0.at[pl.ds(0, rem)],
                        agg_sh.at[pl.ds(s * zrows + (zrows // CH) * CH, rem)])
    plsc.subcore_barrier()

    e0 = wid * ((nch + 2) * CH)

    def _sload(j, t, sem):
        return pltpu.async_copy(src_hbm.at[pl.ds(e0 + j * CH, CH)],
                                sidx[t], sem)

    def _dload(j, t, sem):
        return pltpu.async_copy(dst_hbm.at[pl.ds(e0 + j * CH, CH)],
                                didx[t], sem)

    def _gissue(t):
        return pltpu.async_copy(h_hbm.at[sidx[t]], msg[t], gsem[t])

    def _gwait(t):
        pltpu.make_async_copy(h_hbm.at[sidx[t]], msg[t], gsem[t]).wait()

    def _sissue(t):
        return pltpu.async_copy(msg[t], agg_sh.at[didx[t]], ssem[t],
                                add=True)

    def _swait(t):
        pltpu.make_async_copy(msg[t], agg_sh.at[didx[t]], ssem[t]).wait()

    def _iwait(t):
        pltpu.make_async_copy(src_hbm.at[pl.ds(e0, CH)], sidx[t],
                              isem[t]).wait()

    def _dwait(t):
        pltpu.make_async_copy(dst_hbm.at[pl.ds(e0, CH)], didx[t],
                              dsem[t]).wait()

    pltpu.sync_copy(src_hbm.at[pl.ds(e0, CH)], sidx[0])
    pltpu.sync_copy(dst_hbm.at[pl.ds(e0 + CH, CH)], didx[1])
    _sload(1, 1, is1)
    _dload(0, 0, id0)
    _gissue(0)
    _sissue(1)

    def _pair(p, _):
        for t in (0, 1):
            j = 2 * p + t
            _gwait(t)
            _sload(j + 2, t, isem[t])
            _swait(1 - t)
            _dload(j + 1, 1 - t, dsem[1 - t])
            _iwait(1 - t)
            _gissue(1 - t)
            _dwait(t)
            _sissue(t)
        return 0

    lax.fori_loop(0, nch // 2, _pair, 0)

    _gwait(0)
    _swait(1)
    _iwait(1)
    _dwait(0)
    plsc.subcore_barrier()

    pltpu.sync_copy(agg_sh.at[pl.ds(s * zrows, zrows)],
                    out_hbm.at[c, pl.ds(s * zrows, zrows)])


@functools.partial(jax.jit, static_argnums=(3,))
def _sc_agg(h, src_flat, dst_flat, n_nodes):
    nch = src_flat.shape[0] // (NW * CH) - 2
    agg_rows = ((n_nodes + 1 + NS * 8 - 1) // (NS * 8)) * (NS * 8)
    mesh = plsc.VectorSubcoreMesh(core_axis_name="c", subcore_axis_name="s",
                                  num_cores=NC, num_subcores=NS)
    body = functools.partial(_sc_agg_body, nch, agg_rows)
    kern = pl.kernel(
        body,
        out_type=jax.ShapeDtypeStruct((NC, agg_rows, D), jnp.float32),
        mesh=mesh,
        scratch_types=[
            pltpu.VMEM((CH,), jnp.int32),
            pltpu.VMEM((CH,), jnp.int32),
            pltpu.VMEM((CH,), jnp.int32),
            pltpu.VMEM((CH,), jnp.int32),
            pltpu.VMEM((CH, D), jnp.float32),
            pltpu.VMEM((CH, D), jnp.float32),
            pltpu.VMEM_SHARED((agg_rows, D), jnp.float32),
            pltpu.SemaphoreType.DMA,
            pltpu.SemaphoreType.DMA,
            pltpu.SemaphoreType.DMA,
            pltpu.SemaphoreType.DMA,
            pltpu.SemaphoreType.DMA,
            pltpu.SemaphoreType.DMA,
            pltpu.SemaphoreType.DMA,
            pltpu.SemaphoreType.DMA,
        ],
    )
    return kern(h, src_flat, dst_flat)


def _tc_body(h_ref, a0_ref, a1_ref, w_ref, o_ref):
    agg = a0_ref[0] + a1_ref[0]
    t = jnp.dot(agg, w_ref[...], preferred_element_type=jnp.float32)
    o_ref[...] = h_ref[...] + jnp.maximum(t, 0.0)


def _tc_update(h, agg2, w):
    n = h.shape[0]
    blk = 1000
    grid = (n // blk,)
    return pl.pallas_call(
        _tc_body,
        grid=grid,
        in_specs=[
            pl.BlockSpec((blk, D), lambda i: (i, 0)),
            pl.BlockSpec((1, blk, D), lambda i: (0, i, 0)),
            pl.BlockSpec((1, blk, D), lambda i: (1, i, 0)),
            pl.BlockSpec((D, D), lambda i: (0, 0)),
        ],
        out_specs=pl.BlockSpec((blk, D), lambda i: (i, 0)),
        out_shape=jax.ShapeDtypeStruct((n, D), jnp.float32),
    )(h, agg2, agg2, w)


def kernel(x, edge_index, W):
    n = x.shape[0]
    e = edge_index.shape[1]
    src = edge_index[0].astype(jnp.int32)
    dst = edge_index[1].astype(jnp.int32)

    q = (e + NW - 1) // NW
    q_pad = (q + 2 * CH - 1) // (2 * CH) * (2 * CH)
    src_m = jnp.concatenate(
        [src, jnp.zeros((q_pad * NW - e,), jnp.int32)]).reshape(NW, q_pad)
    dst_m = jnp.concatenate(
        [dst, jnp.full((q_pad * NW - e,), n, jnp.int32)]).reshape(NW, q_pad)
    zpad = jnp.zeros((NW, 2 * CH), jnp.int32)
    npad = jnp.full((NW, 2 * CH), n, jnp.int32)
    src_flat = jnp.concatenate([zpad, src_m, zpad], axis=1).reshape(-1)
    dst_flat = jnp.concatenate([npad, dst_m, npad], axis=1).reshape(-1)

    h = x
    for l in range(W.shape[0]):
        agg2 = _sc_agg(h, src_flat, dst_flat, n)
        h = _tc_update(h, agg2, W[l])
    return h


# speedup vs baseline: 2.7603x; 2.7603x over previous
import functools

import jax
import jax.numpy as jnp
from jax import lax
from jax.experimental import pallas as pl
from jax.experimental.pallas import tpu as pltpu
from jax.experimental.pallas import tpu_sc as plsc

NC = 2
NS = 16
NW = NC * NS
CH = 128
D = 128


def _sc_agg_body(nch, agg_rows,
                 h_hbm, idx_hbm, out_hbm,
                 idx_v, msg, agg_sh, gsem):
    c = lax.axis_index("c")
    s = lax.axis_index("s")
    wid = c * NS + s

    zrows = agg_rows // NS

    def _zero_row(i, _):
        for j in range(D // 16):
            msg[i, pl.ds(j * 16, 16)] = jnp.zeros((16,), jnp.float32)
        return 0

    lax.fori_loop(0, CH, _zero_row, 0)
    for k in range(zrows // CH):
        pltpu.sync_copy(msg, agg_sh.at[pl.ds(s * zrows + k * CH, CH)])
    rem = zrows % CH
    if rem:
        pltpu.sync_copy(msg.at[pl.ds(0, rem)],
                        agg_sh.at[pl.ds(s * zrows + (zrows // CH) * CH, rem)])

    pltpu.sync_copy(idx_hbm.at[pl.ds(wid * nch, nch)], idx_v)
    plsc.subcore_barrier()

    def _chunk(j, _):
        pltpu.async_copy(h_hbm.at[idx_v.at[j, 0]], msg, gsem).wait()
        pltpu.sync_copy(msg, agg_sh.at[idx_v.at[j, 1]], add=True)
        return 0

    lax.fori_loop(0, nch, _chunk, 0)
    plsc.subcore_barrier()

    pltpu.sync_copy(agg_sh.at[pl.ds(s * zrows, zrows)],
                    out_hbm.at[c, pl.ds(s * zrows, zrows)])


@functools.partial(jax.jit, static_argnums=(2,))
def _sc_agg(h, idx, n_nodes):
    nch = idx.shape[0] // NW
    agg_rows = ((n_nodes + 1 + NS * 8 - 1) // (NS * 8)) * (NS * 8)
    mesh = plsc.VectorSubcoreMesh(core_axis_name="c", subcore_axis_name="s",
                                  num_cores=NC, num_subcores=NS)
    body = functools.partial(_sc_agg_body, nch, agg_rows)
    kern = pl.kernel(
        body,
        out_type=jax.ShapeDtypeStruct((NC, agg_rows, D), jnp.float32),
        mesh=mesh,
        scratch_types=[
            pltpu.VMEM((nch, 2, CH), jnp.int32),
            pltpu.VMEM((CH, D), jnp.float32),
            pltpu.VMEM_SHARED((agg_rows, D), jnp.float32),
            pltpu.SemaphoreType.DMA,
        ],
    )
    return kern(h, idx)


def _tc_body(h_ref, a0_ref, a1_ref, w_ref, o_ref):
    agg = a0_ref[0] + a1_ref[0]
    t = jnp.dot(agg, w_ref[...], preferred_element_type=jnp.float32)
    o_ref[...] = h_ref[...] + jnp.maximum(t, 0.0)


def _tc_update(h, agg2, w):
    n = h.shape[0]
    blk = 1000
    grid = (n // blk,)
    return pl.pallas_call(
        _tc_body,
        grid=grid,
        in_specs=[
            pl.BlockSpec((blk, D), lambda i: (i, 0)),
            pl.BlockSpec((1, blk, D), lambda i: (0, i, 0)),
            pl.BlockSpec((1, blk, D), lambda i: (1, i, 0)),
            pl.BlockSpec((D, D), lambda i: (0, 0)),
        ],
        out_specs=pl.BlockSpec((blk, D), lambda i: (i, 0)),
        out_shape=jax.ShapeDtypeStruct((n, D), jnp.float32),
    )(h, agg2, agg2, w)


def kernel(x, edge_index, W):
    n = x.shape[0]
    e = edge_index.shape[1]
    src = edge_index[0].astype(jnp.int32)
    dst = edge_index[1].astype(jnp.int32)

    q = (e + NW - 1) // NW
    q_pad = (q + CH - 1) // CH * CH
    src_m = jnp.concatenate(
        [src, jnp.zeros((q_pad * NW - e,), jnp.int32)]).reshape(-1, CH)
    dst_m = jnp.concatenate(
        [dst, jnp.full((q_pad * NW - e,), n, jnp.int32)]).reshape(-1, CH)
    idx = jnp.stack([src_m, dst_m], axis=1)

    h = x
    for l in range(W.shape[0]):
        agg2 = _sc_agg(h, idx, n)
        h = _tc_update(h, agg2, W[l])
    return h
